# Initial kernel scaffold; baseline (speedup 1.0000x reference)
import jax
import jax.numpy as jnp
from jax.experimental import pallas as pl


def kernel(x, in_degree, out_degree, atom_table, in_table, out_table, graph_token_w):
    raise NotImplementedError("write your pallas kernel here")


# speedup vs baseline: 14.2962x; 14.2962x over previous
import functools

import jax
import jax.numpy as jnp
from jax import lax
from jax.experimental import pallas as pl
from jax.experimental.pallas import tpu as pltpu
from jax.experimental.pallas import tpu_sc as plsc

N_GRAPH, N_NODE, N_FEAT = 1024, 128, 9
HIDDEN = 64
NUM_WORKERS = 32
GRAPHS_PER_WORKER = N_GRAPH // NUM_WORKERS
LANES = 16
VPR = HIDDEN // LANES


def _sc_kernel():
    mesh = plsc.VectorSubcoreMesh(core_axis_name="c", subcore_axis_name="s")

    @functools.partial(
        pl.kernel,
        mesh=mesh,
        out_type=jax.ShapeDtypeStruct((N_GRAPH, N_NODE + 1, HIDDEN), jnp.float32),
        scratch_types=[
            pltpu.VMEM((N_FEAT, N_NODE), jnp.int32),
            pltpu.VMEM((N_NODE,), jnp.int32),
            pltpu.VMEM((N_NODE,), jnp.int32),
            pltpu.VMEM((N_FEAT, N_NODE, HIDDEN), jnp.float32),
            pltpu.VMEM((2, N_NODE, HIDDEN), jnp.float32),
            pltpu.VMEM((N_NODE + 1, HIDDEN), jnp.float32),
            pltpu.SemaphoreType.DMA,
        ],
        compiler_params=pltpu.CompilerParams(use_tc_tiling_on_sc=False),
    )
    def k(x_t, in_deg, out_deg, atom_t, in_t, out_t, token,
          out, idxa, idxi, idxo, rows, drows, obuf, sem):
        wid = lax.axis_index("s") * 2 + lax.axis_index("c")

        pltpu.sync_copy(token, obuf.at[pl.ds(0, 1)])

        def per_graph(i, carry):
            g = wid * GRAPHS_PER_WORKER + i
            pltpu.sync_copy(x_t.at[g], idxa)
            pltpu.sync_copy(in_deg.at[g], idxi)
            pltpu.sync_copy(out_deg.at[g], idxo)
            cps = []
            for j in range(N_FEAT):
                cps.append(pltpu.async_copy(atom_t.at[idxa.at[j]], rows.at[j], sem))
            cps.append(pltpu.async_copy(in_t.at[idxi], drows.at[0], sem))
            cps.append(pltpu.async_copy(out_t.at[idxo], drows.at[1], sem))
            for c in cps:
                c.wait()

            def per_node(n, nc):
                for h in range(VPR):
                    sl = pl.ds(h * LANES, LANES)
                    acc = rows[0, n, sl]
                    for j in range(1, N_FEAT):
                        acc = acc + rows[j, n, sl]
                    acc = acc + drows[0, n, sl] + drows[1, n, sl]
                    obuf[1 + n, sl] = acc
                return nc

            lax.fori_loop(0, N_NODE, per_node, 0)
            pltpu.sync_copy(obuf, out.at[g])
            return carry

        lax.fori_loop(0, GRAPHS_PER_WORKER, per_graph, 0)

    return k


def kernel(x, in_degree, out_degree, atom_table, in_table, out_table, graph_token_w):
    x_t = jnp.transpose(x.astype(jnp.int32), (0, 2, 1))
    return _sc_kernel()(
        x_t,
        in_degree.astype(jnp.int32),
        out_degree.astype(jnp.int32),
        atom_table,
        in_table,
        out_table,
        graph_token_w,
    )
